# baseline (speedup 1.0000x reference)
import functools

import jax
import jax.numpy as jnp
from jax.experimental import pallas as pl
from jax.experimental.pallas import tpu as pltpu


def _cdiv(a, b):
    return (a + b - 1) // b


def _round_up(x, m):
    return _cdiv(x, m) * m


def _sploss_kernel(pairs_ref, vt_ref, outp_ref, outc_ref,
                   *, m_tile, unroll, delta):
    t = pl.program_id(0)

    @pl.when(t == 0)
    def _():
        outp_ref[...] = jnp.zeros_like(outp_ref)
        outc_ref[...] = jnp.zeros_like(outc_ref)

    def chunk(u, carry):
        acc_p, acc_c = carry
        base = u * unroll
        d2s = []
        for k in range(unroll):
            m = base + k
            w = pairs_ref[0, 0, m]
            ii = w & 0xFFFF
            jj = jax.lax.shift_right_logical(w, 16)
            d = vt_ref[ii] - vt_ref[jj]
            r = d * d
            d2s.append(r[:, 0:128] + r[:, 128:256] + r[:, 256:384])
        for g in range(unroll // 8):
            d2 = jnp.concatenate(d2s[8 * g:8 * g + 8], axis=0)
            hit = d2 < (delta * delta)
            val = delta - jnp.sqrt(d2)
            p = jnp.where(hit, val * val, 0.0)
            cnt = jnp.where(p > 0.0, 1.0, 0.0)
            acc_p = acc_p + p
            acc_c = acc_c + cnt
        return acc_p, acc_c

    zero = jnp.zeros((8, 128), jnp.float32)
    acc_p, acc_c = jax.lax.fori_loop(0, m_tile // unroll, chunk, (zero, zero))
    outp_ref[0] = outp_ref[0] + acc_p
    outc_ref[0] = outc_ref[0] + acc_c


def kernel(vertices, pairs):
    delta = 1.5
    eps = 1e-6
    vertices = jnp.asarray(vertices, jnp.float32)
    pairs = jnp.asarray(pairs, jnp.int32)
    B, N, C = vertices.shape
    P = pairs.shape[0]
    if P == 0:
        return jnp.float32(0.0)

    m_tile = 512
    unroll = 512
    nc = 1

    total_tiles = _round_up(_cdiv(P, m_tile), nc)
    p_pad = total_tiles * m_tile
    npad = p_pad - P
    if npad:
        pairs = jnp.pad(pairs, ((0, npad), (0, 0)))
    packed = pairs[:, 0] | (pairs[:, 1] << 16)
    pairs_rows = packed.reshape(total_tiles, 1, m_tile)

    vt = jnp.transpose(vertices, (1, 2, 0)).reshape(N, 1, C * B)

    kfn = functools.partial(_sploss_kernel, m_tile=m_tile, unroll=unroll,
                            delta=delta)
    outp, outc = pl.pallas_call(
        kfn,
        out_shape=(jax.ShapeDtypeStruct((nc, 8, 128), jnp.float32),
                   jax.ShapeDtypeStruct((nc, 8, 128), jnp.float32)),
        grid=(total_tiles,),
        in_specs=[
            pl.BlockSpec((1, 1, m_tile),
                         lambda t: (t, 0, 0),
                         memory_space=pltpu.SMEM),
            pl.BlockSpec((N, 1, C * B), lambda t: (0, 0, 0)),
        ],
        out_specs=(pl.BlockSpec((1, 8, 128), lambda t: (0, 0, 0)),
                   pl.BlockSpec((1, 8, 128), lambda t: (0, 0, 0))),
        compiler_params=pltpu.CompilerParams(
            dimension_semantics=("arbitrary",),
            vmem_limit_bytes=60 * 1024 * 1024),
    )(pairs_rows, vt)

    sum_p = jnp.sum(outp) - jnp.float32(npad * B) * (delta * delta)
    cnt = jnp.sum(outc) - jnp.float32(npad * B)
    return sum_p / (cnt + eps)


# speedup vs baseline: 1.2341x; 1.2341x over previous
import functools

import jax
import jax.numpy as jnp
from jax.experimental import pallas as pl
from jax.experimental.pallas import tpu as pltpu


def _cdiv(a, b):
    return (a + b - 1) // b


def _round_up(x, m):
    return _cdiv(x, m) * m


def _sploss_kernel(pairs_ref, vt_ref, outp_ref, outc_ref,
                   *, m_tile, unroll, delta):
    t = pl.program_id(0)

    @pl.when(t == 0)
    def _():
        outp_ref[...] = jnp.zeros_like(outp_ref)
        outc_ref[...] = jnp.zeros_like(outc_ref)

    def chunk(u, carry):
        acc_p, acc_c = carry
        base = u * unroll
        d2s = []
        for k in range(unroll):
            m = base + k
            w = pairs_ref[0, 0, m]
            ii = w & 0xFFFF
            jj = jax.lax.shift_right_logical(w, 16)
            d = vt_ref[ii] - vt_ref[jj]
            r = d * d
            d2s.append(r[:, 0:128] + r[:, 128:256] + r[:, 256:384])
        for g in range(unroll // 8):
            d2 = jnp.concatenate(d2s[8 * g:8 * g + 8], axis=0)
            hit = d2 < (delta * delta)
            val = delta - jnp.sqrt(d2)
            p = jnp.where(hit, val * val, 0.0)
            cnt = jnp.where(p > 0.0, 1.0, 0.0)
            acc_p = acc_p + p
            acc_c = acc_c + cnt
        return acc_p, acc_c

    zero = jnp.zeros((8, 128), jnp.float32)
    acc_p, acc_c = jax.lax.fori_loop(0, m_tile // unroll, chunk, (zero, zero))
    outp_ref[0] = outp_ref[0] + acc_p
    outc_ref[0] = outc_ref[0] + acc_c


def kernel(vertices, pairs):
    delta = 1.5
    eps = 1e-6
    vertices = jnp.asarray(vertices, jnp.float32)
    pairs = jnp.asarray(pairs, jnp.int32)
    B, N, C = vertices.shape
    P = pairs.shape[0]
    if P == 0:
        return jnp.float32(0.0)

    m_tile = 4096
    unroll = 32
    nc = 1

    total_tiles = _round_up(_cdiv(P, m_tile), nc)
    p_pad = total_tiles * m_tile
    npad = p_pad - P
    if npad:
        pairs = jnp.pad(pairs, ((0, npad), (0, 0)))
    packed = pairs[:, 0] | (pairs[:, 1] << 16)
    pairs_rows = packed.reshape(total_tiles, 1, m_tile)

    vt = jnp.transpose(vertices, (1, 2, 0)).reshape(N, 1, C * B)

    kfn = functools.partial(_sploss_kernel, m_tile=m_tile, unroll=unroll,
                            delta=delta)
    outp, outc = pl.pallas_call(
        kfn,
        out_shape=(jax.ShapeDtypeStruct((nc, 8, 128), jnp.float32),
                   jax.ShapeDtypeStruct((nc, 8, 128), jnp.float32)),
        grid=(total_tiles,),
        in_specs=[
            pl.BlockSpec((1, 1, m_tile),
                         lambda t: (t, 0, 0),
                         memory_space=pltpu.SMEM),
            pl.BlockSpec((N, 1, C * B), lambda t: (0, 0, 0)),
        ],
        out_specs=(pl.BlockSpec((1, 8, 128), lambda t: (0, 0, 0)),
                   pl.BlockSpec((1, 8, 128), lambda t: (0, 0, 0))),
        compiler_params=pltpu.CompilerParams(
            dimension_semantics=("arbitrary",),
            vmem_limit_bytes=60 * 1024 * 1024),
    )(pairs_rows, vt)

    sum_p = jnp.sum(outp) - jnp.float32(npad * B) * (delta * delta)
    cnt = jnp.sum(outc) - jnp.float32(npad * B)
    return sum_p / (cnt + eps)


# speedup vs baseline: 1.2401x; 1.0049x over previous
import functools

import jax
import jax.numpy as jnp
from jax.experimental import pallas as pl
from jax.experimental.pallas import tpu as pltpu


def _cdiv(a, b):
    return (a + b - 1) // b


def _round_up(x, m):
    return _cdiv(x, m) * m


def _sploss_kernel(pairs_ref, vt_ref, outp_ref, outc_ref,
                   *, m_tile, unroll, delta):
    t = pl.program_id(0)

    @pl.when(t == 0)
    def _():
        outp_ref[...] = jnp.zeros_like(outp_ref)
        outc_ref[...] = jnp.zeros_like(outc_ref)

    def chunk(u, carry):
        acc_p, acc_c = carry
        base = u * unroll
        d2s = []
        for k in range(unroll):
            m = base + k
            w = pairs_ref[0, 0, m]
            ii = w & 0xFFFF
            jj = jax.lax.shift_right_logical(w, 16)
            d = vt_ref[ii] - vt_ref[jj]
            r = d * d
            d2s.append(r[:, 0:128] + r[:, 128:256] + r[:, 256:384])
        for g in range(unroll // 8):
            d2 = jnp.concatenate(d2s[8 * g:8 * g + 8], axis=0)
            hit = d2 < (delta * delta)
            d = d2 * jax.lax.rsqrt(jnp.maximum(d2, 1e-30))
            val = delta - d
            p = jnp.where(hit, val * val, 0.0)
            cnt = jnp.where(p > 0.0, 1.0, 0.0)
            acc_p = acc_p + p
            acc_c = acc_c + cnt
        return acc_p, acc_c

    zero = jnp.zeros((8, 128), jnp.float32)
    acc_p, acc_c = jax.lax.fori_loop(0, m_tile // unroll, chunk, (zero, zero))
    outp_ref[0] = outp_ref[0] + acc_p
    outc_ref[0] = outc_ref[0] + acc_c


def kernel(vertices, pairs):
    delta = 1.5
    eps = 1e-6
    vertices = jnp.asarray(vertices, jnp.float32)
    pairs = jnp.asarray(pairs, jnp.int32)
    B, N, C = vertices.shape
    P = pairs.shape[0]
    if P == 0:
        return jnp.float32(0.0)

    m_tile = 4096
    unroll = 32
    nc = 1

    total_tiles = _round_up(_cdiv(P, m_tile), nc)
    p_pad = total_tiles * m_tile
    npad = p_pad - P
    if npad:
        pairs = jnp.pad(pairs, ((0, npad), (0, 0)))
    packed = pairs[:, 0] | (pairs[:, 1] << 16)
    pairs_rows = packed.reshape(total_tiles, 1, m_tile)

    vt = jnp.transpose(vertices, (1, 2, 0)).reshape(N, 1, C * B)

    kfn = functools.partial(_sploss_kernel, m_tile=m_tile, unroll=unroll,
                            delta=delta)
    outp, outc = pl.pallas_call(
        kfn,
        out_shape=(jax.ShapeDtypeStruct((nc, 8, 128), jnp.float32),
                   jax.ShapeDtypeStruct((nc, 8, 128), jnp.float32)),
        grid=(total_tiles,),
        in_specs=[
            pl.BlockSpec((1, 1, m_tile),
                         lambda t: (t, 0, 0),
                         memory_space=pltpu.SMEM),
            pl.BlockSpec((N, 1, C * B), lambda t: (0, 0, 0)),
        ],
        out_specs=(pl.BlockSpec((1, 8, 128), lambda t: (0, 0, 0)),
                   pl.BlockSpec((1, 8, 128), lambda t: (0, 0, 0))),
        compiler_params=pltpu.CompilerParams(
            dimension_semantics=("arbitrary",),
            vmem_limit_bytes=60 * 1024 * 1024),
    )(pairs_rows, vt)

    sum_p = jnp.sum(outp) - jnp.float32(npad * B) * (delta * delta)
    cnt = jnp.sum(outc) - jnp.float32(npad * B)
    return sum_p / (cnt + eps)
